# Initial kernel scaffold; baseline (speedup 1.0000x reference)
import jax
import jax.numpy as jnp
from jax.experimental import pallas as pl


def kernel(x, adj, num_graphs, in_batch, cluster, params):
    raise NotImplementedError("write your pallas kernel here")


# speedup vs baseline: 8.0597x; 8.0597x over previous
import functools

import jax
import jax.numpy as jnp
from jax import lax
from jax.experimental import pallas as pl
from jax.experimental.pallas import tpu as pltpu
from jax.experimental.pallas import tpu_sc as plsc

N = 10000
E = 320000
D = 128
GNN = 1000
CN = 100
C = 1000
CC = C * C
EPS = 1e-5

NC = 2
NS = 16
NW = NC * NS
ERP = 2560
RPW = ERP // NW
RPC = ERP // NS
EPAD = ERP * 128 - E
NP = N + 16
RT = 624
HN = N // NC
HRT = 312
QCC = CC // 4
CH = 40

_f32 = jnp.float32
_i32 = jnp.int32

_sc_mesh = plsc.VectorSubcoreMesh(
    core_axis_name="c", subcore_axis_name="s", num_cores=NC, num_subcores=NS)


def _elu(v):
    return jnp.where(v > 0, v, jnp.exp(v) - 1.0)


@functools.partial(
    pl.kernel,
    out_type=(jax.ShapeDtypeStruct((N,), _f32),
              jax.ShapeDtypeStruct((CC,), _f32)),
    mesh=_sc_mesh,
    scratch_types=[
        pltpu.VMEM_SHARED((HN + 16,), _f32),
        pltpu.VMEM_SHARED((QCC + 16,), _f32),
        pltpu.VMEM((RPC, 128), _i32),
        pltpu.VMEM((RPC, 128), _i32),
        pltpu.VMEM((RPC, 128), _i32),
        pltpu.VMEM((RPC, 128), _i32),
        pltpu.VMEM((128,), _f32),
        pltpu.VMEM((2000,), _f32),
    ],
)
def _sc_counts(src2d, dst2d, zeros1d, deg_out, acnt_out,
               deg_sh, acnt_sh, sbuf2d, dbuf2d, pbuf2d, qbuf2d, ones_b,
               zbuf):
    cid = lax.axis_index("c")
    sid = lax.axis_index("s")

    pltpu.sync_copy(zeros1d.at[pl.ds(0, 2000)], zbuf)

    doff = pl.multiple_of(sid * HRT, 8)
    pltpu.sync_copy(zbuf.at[pl.ds(0, HRT)], deg_sh.at[pl.ds(doff, HRT)])

    @pl.when(sid == 0)
    def _():
        pltpu.sync_copy(zbuf.at[pl.ds(0, HN - NS * HRT)],
                        deg_sh.at[pl.ds(NS * HRT, HN - NS * HRT)])

    def _zero_acnt():
        @pl.loop(0, 8)
        def _(t):
            ci = t * NS + sid

            @pl.when(ci < 125)
            def _():
                pltpu.sync_copy(zbuf, acnt_sh.at[pl.ds(ci * 2000, 2000)])

    _zero_acnt()

    for l in range(8):
        ones_b[pl.ds(l * 16, 16)] = jnp.ones((16,), _f32)

    base = pl.multiple_of(sid * RPC, 8)
    pltpu.sync_copy(src2d.at[pl.ds(base, RPC)], sbuf2d)
    pltpu.sync_copy(dst2d.at[pl.ds(base, RPC)], dbuf2d)

    cbase_n = cid * HN

    @pl.loop(0, RPC)
    def _(j):
        for l in range(8):
            sv = sbuf2d[j, pl.ds(l * 16, 16)]
            dv = dbuf2d[j, pl.ds(l * 16, 16)]
            cs = lax.div(sv, GNN) * CN + lax.rem(sv, CN)
            cd = lax.div(dv, GNN) * CN + lax.rem(dv, CN)
            pbuf2d[j, pl.ds(l * 16, 16)] = cd * C + cs
            ld = dv - cbase_n
            dbuf2d[j, pl.ds(l * 16, 16)] = jnp.where(
                (ld >= 0) & (ld < HN), ld, HN)

    def _rebase(qbase):
        @pl.loop(0, RPC)
        def _(j):
            for l in range(8):
                lp = pbuf2d[j, pl.ds(l * 16, 16)] - qbase
                qbuf2d[j, pl.ds(l * 16, 16)] = jnp.where(
                    (lp >= 0) & (lp < QCC), lp, QCC)

    def _acnt_writeback(qbase):
        @pl.loop(0, 8)
        def _(t):
            ci = t * NS + sid

            @pl.when(ci < 125)
            def _():
                pltpu.sync_copy(acnt_sh.at[pl.ds(ci * 2000, 2000)], zbuf)
                pltpu.sync_copy(zbuf,
                                acnt_out.at[pl.ds(qbase + ci * 2000, 2000)])

    _rebase(cid * QCC)
    plsc.subcore_barrier()

    @pl.loop(0, RPC)
    def _(j):
        pltpu.sync_copy(ones_b, deg_sh.at[dbuf2d.at[j]], add=True)
        pltpu.sync_copy(ones_b, acnt_sh.at[qbuf2d.at[j]], add=True)

    plsc.subcore_barrier()

    pltpu.sync_copy(deg_sh.at[pl.ds(doff, HRT)], zbuf.at[pl.ds(0, HRT)])
    pltpu.sync_copy(zbuf.at[pl.ds(0, HRT)],
                    deg_out.at[pl.ds(cbase_n + doff, HRT)])

    @pl.when(sid == 0)
    def _():
        pltpu.sync_copy(deg_sh.at[pl.ds(NS * HRT, HN - NS * HRT)],
                        zbuf.at[pl.ds(HRT, HN - NS * HRT)])
        pltpu.sync_copy(zbuf.at[pl.ds(HRT, HN - NS * HRT)],
                        deg_out.at[pl.ds(cbase_n + NS * HRT, HN - NS * HRT)])

    _acnt_writeback(cid * QCC)
    plsc.subcore_barrier()

    pltpu.sync_copy(zeros1d.at[pl.ds(0, 2000)], zbuf)
    _zero_acnt()
    _rebase((2 + cid) * QCC)
    plsc.subcore_barrier()

    @pl.loop(0, RPC)
    def _(j):
        pltpu.sync_copy(ones_b, acnt_sh.at[qbuf2d.at[j]], add=True)

    plsc.subcore_barrier()
    _acnt_writeback((2 + cid) * QCC)


@functools.partial(
    pl.kernel,
    out_type=jax.ShapeDtypeStruct((NC, N, D), _f32),
    mesh=_sc_mesh,
    scratch_types=[
        pltpu.VMEM_SHARED((NP, D), _f32),
        pltpu.VMEM((CH, 128), _i32),
        pltpu.VMEM((CH, 128), _i32),
        pltpu.VMEM((128, D), _f32),
        pltpu.VMEM((128, D), _f32),
        pltpu.SemaphoreType.DMA,
        pltpu.SemaphoreType.DMA,
    ],
)
def _sc_scatter(g_hbm, src2d, dst2d, zrows, acc_out,
                acc_sh, sbuf2d, dbuf2d, rows0, rows1, semg0, semg1):
    cid = lax.axis_index("c")
    sid = lax.axis_index("s")
    wid = sid * NC + cid

    base = pl.multiple_of(wid * RPW, 8)
    off = pl.multiple_of(sid * RT, 8)
    rows = (rows0, rows1)
    semg = (semg0, semg1)

    pltpu.sync_copy(zrows, rows0)

    @pl.loop(0, 4)
    def _(q):
        pltpu.sync_copy(rows0, acc_sh.at[pl.ds(off + q * 128, 128)])

    pltpu.sync_copy(rows0.at[pl.ds(0, RT - 512)],
                    acc_sh.at[pl.ds(off + 512, RT - 512)])

    @pl.when(sid == 0)
    def _():
        pltpu.sync_copy(rows0.at[pl.ds(0, N - NS * RT)],
                        acc_sh.at[pl.ds(NS * RT, N - NS * RT)])

    plsc.subcore_barrier()

    for c in range(RPW // CH):
        cb = pl.multiple_of(base + c * CH, 8)
        pltpu.sync_copy(src2d.at[pl.ds(cb, CH)], sbuf2d)
        pltpu.sync_copy(dst2d.at[pl.ds(cb, CH)], dbuf2d)

        pltpu.async_copy(g_hbm.at[sbuf2d.at[0]], rows0, semg0)

        @pl.loop(0, CH // 2)
        def _(t):
            for b in range(2):
                j = t * 2 + b
                pltpu.make_async_copy(g_hbm.at[sbuf2d.at[j]], rows[b],
                                      semg[b]).wait()

                @pl.when(j + 1 < CH)
                def _():
                    pltpu.async_copy(g_hbm.at[sbuf2d.at[j + 1]], rows[1 - b],
                                     semg[1 - b])

                pltpu.sync_copy(rows[b], acc_sh.at[dbuf2d.at[j]], add=True)

    plsc.subcore_barrier()

    pltpu.sync_copy(acc_sh.at[pl.ds(off, RT)],
                    acc_out.at[cid, pl.ds(off, RT)])

    @pl.when(sid == 0)
    def _():
        pltpu.sync_copy(acc_sh.at[pl.ds(NS * RT, N - NS * RT)],
                        acc_out.at[cid, pl.ds(NS * RT, N - NS * RT)])


_GB = 1000


def _acc_sum(ap_ref):
    return ap_ref[0] + ap_ref[1]


def _tc_prep_body(x_ref, w_ref, dis_ref, g_ref):
    g_ref[...] = dis_ref[...] * jnp.dot(x_ref[...], w_ref[...],
                                        preferred_element_type=_f32)


_tc_prep = pl.pallas_call(
    _tc_prep_body,
    grid=(N // _GB,),
    in_specs=[
        pl.BlockSpec((_GB, D), lambda i: (i, 0)),
        pl.BlockSpec((D, D), lambda i: (0, 0)),
        pl.BlockSpec((_GB, 1), lambda i: (i, 0)),
    ],
    out_specs=pl.BlockSpec((_GB, D), lambda i: (i, 0)),
    out_shape=jax.ShapeDtypeStruct((N, D), _f32),
)


def _tc_layer_body(ap_ref, g_ref, dis_ref, b_ref, w_ref, o_ref):
    dis = dis_ref[...]
    g = g_ref[...]
    h = dis * (_acc_sum(ap_ref) + g) + b_ref[...]
    h = _elu(h)
    o_ref[...] = dis * jnp.dot(h, w_ref[...], preferred_element_type=_f32)


_tc_layer = pl.pallas_call(
    _tc_layer_body,
    grid=(N // _GB,),
    in_specs=[
        pl.BlockSpec((NC, _GB, D), lambda i: (0, i, 0)),
        pl.BlockSpec((_GB, D), lambda i: (i, 0)),
        pl.BlockSpec((_GB, 1), lambda i: (i, 0)),
        pl.BlockSpec((1, D), lambda i: (0, 0)),
        pl.BlockSpec((D, D), lambda i: (0, 0)),
    ],
    out_specs=pl.BlockSpec((_GB, D), lambda i: (i, 0)),
    out_shape=jax.ShapeDtypeStruct((N, D), _f32),
)


def _tc_post_body(ap_ref, g_ref, dis_ref, b3_ref, win0_ref, bin0_ref,
                  win1_ref, bin1_ref, wt1_ref, bt1_ref, wt2_ref, bt2_ref,
                  z_ref):
    dis = dis_ref[...]
    h = dis * (_acc_sum(ap_ref) + g_ref[...]) + b3_ref[...]
    h = _elu(jnp.dot(h, win0_ref[...], preferred_element_type=_f32)
             + bin0_ref[...])
    h = _elu(jnp.dot(h, win1_ref[...], preferred_element_type=_f32)
             + bin1_ref[...])
    mean = jnp.mean(h, axis=0, keepdims=True)
    var = jnp.mean(h * h, axis=0, keepdims=True) - mean * mean
    y = (h - mean) * lax.rsqrt(var + EPS)
    xp = jnp.mean(y.reshape(GNN // CN, CN, D), axis=0)
    z = _elu(jnp.dot(xp, wt1_ref[...], preferred_element_type=_f32)
             + bt1_ref[...])
    z_ref[0] = _elu(jnp.dot(z, wt2_ref[...], preferred_element_type=_f32)
                    + bt2_ref[...])


_tc_post = pl.pallas_call(
    _tc_post_body,
    grid=(N // _GB,),
    in_specs=[
        pl.BlockSpec((NC, _GB, D), lambda i: (0, i, 0)),
        pl.BlockSpec((_GB, D), lambda i: (i, 0)),
        pl.BlockSpec((_GB, 1), lambda i: (i, 0)),
        pl.BlockSpec((1, D), lambda i: (0, 0)),
        pl.BlockSpec((D, D), lambda i: (0, 0)),
        pl.BlockSpec((1, D), lambda i: (0, 0)),
        pl.BlockSpec((D, D), lambda i: (0, 0)),
        pl.BlockSpec((1, D), lambda i: (0, 0)),
        pl.BlockSpec((D, 128), lambda i: (0, 0)),
        pl.BlockSpec((1, 128), lambda i: (0, 0)),
        pl.BlockSpec((128, 128), lambda i: (0, 0)),
        pl.BlockSpec((1, 128), lambda i: (0, 0)),
    ],
    out_specs=pl.BlockSpec((1, CN, 128), lambda i: (i, 0, 0)),
    out_shape=jax.ShapeDtypeStruct((N // _GB, CN, 128), _f32),
)


def _tc_pooled_body(at_ref, z_ref, w4_ref, b4_ref, w5_ref, b5_ref,
                    wfc1_ref, bfc1_ref, wbv0_ref, bbv0_ref, wbv1_ref,
                    bbv1_ref, o_ref):
    cnt = at_ref[...]
    r = lax.broadcasted_iota(_i32, (C, C), 0)
    cix = lax.broadcasted_iota(_i32, (C, C), 1)
    at = jnp.where((cnt > 0) & (r != cix), 1.0, 0.0).astype(_f32)
    deg2 = jnp.sum(at, axis=1, keepdims=True) + 1.0
    dis2 = lax.rsqrt(deg2)
    ateye = at + jnp.where(r == cix, 1.0, 0.0).astype(_f32)

    def pconv(zz, w, b):
        v = dis2 * jnp.dot(zz, w, preferred_element_type=_f32)
        return dis2 * jnp.dot(ateye, v, preferred_element_type=_f32) + b

    z = _elu(pconv(z_ref[...], w4_ref[...], b4_ref[...]))
    z = _elu(pconv(z, w5_ref[...], b5_ref[...]))
    z = _elu(jnp.dot(z, wfc1_ref[...], preferred_element_type=_f32)
             + bfc1_ref[...])
    z = _elu(jnp.dot(z, wbv0_ref[...], preferred_element_type=_f32)
             + bbv0_ref[...])
    o_ref[...] = _elu(jnp.dot(z, wbv1_ref[...], preferred_element_type=_f32)
                      + bbv1_ref[...])


_tc_pooled = pl.pallas_call(
    _tc_pooled_body,
    out_shape=jax.ShapeDtypeStruct((C, C), _f32),
)


def _tc_head_body(k_ref, wfc2_ref, bfc2_ref, wfc3_ref, bfc3_ref,
                  wav0_ref, bav0_ref, wav1_ref, bav1_ref, o_ref):
    k = _elu(jnp.dot(k_ref[...], wfc2_ref[...], preferred_element_type=_f32)
             + bfc2_ref[...])
    k = jnp.dot(k, wfc3_ref[...], preferred_element_type=_f32) + bfc3_ref[...]
    k = _elu(jnp.dot(k, wav0_ref[...], preferred_element_type=_f32)
             + bav0_ref[...])
    o_ref[...] = _elu(jnp.dot(k, wav1_ref[...], preferred_element_type=_f32)
                      + bav1_ref[...])


_tc_head = pl.pallas_call(
    _tc_head_body,
    grid=(N // _GB,),
    in_specs=[
        pl.BlockSpec((_GB, CN), lambda i: (i, 0)),
        pl.BlockSpec((CN, 256), lambda i: (0, 0)),
        pl.BlockSpec((1, 256), lambda i: (0, 0)),
        pl.BlockSpec((256, 128), lambda i: (0, 0)),
        pl.BlockSpec((1, 128), lambda i: (0, 0)),
        pl.BlockSpec((128, 128), lambda i: (0, 0)),
        pl.BlockSpec((1, 128), lambda i: (0, 0)),
        pl.BlockSpec((128, 128), lambda i: (0, 0)),
        pl.BlockSpec((1, 128), lambda i: (0, 0)),
    ],
    out_specs=pl.BlockSpec((_GB, 128), lambda i: (i, 0)),
    out_shape=jax.ShapeDtypeStruct((N, 128), _f32),
)


def kernel(x, adj, num_graphs, in_batch, cluster, params):
    p = params
    src2d = jnp.concatenate(
        [adj[0].astype(_i32), jnp.zeros((EPAD,), _i32)]).reshape(ERP, 128)
    dst2d = jnp.concatenate(
        [adj[1].astype(_i32), jnp.full((EPAD,), N, _i32)]).reshape(ERP, 128)
    zeros1d = jnp.zeros((8000,), _f32)
    zrows = jnp.zeros((128, D), _f32)

    deg, acnt = _sc_counts(src2d, dst2d, zeros1d)
    dis = lax.rsqrt(deg + 1.0)[:, None]

    def row(b):
        return b.reshape(1, -1)

    def scatter(g):
        return _sc_scatter(g, src2d, dst2d, zrows)

    g = _tc_prep(x, p["W1"], dis)
    accp = scatter(g)
    g = _tc_layer(accp, g, dis, row(p["b1"]), p["W2"])
    accp = scatter(g)
    g = _tc_layer(accp, g, dis, row(p["b2"]), p["W3"])
    accp = scatter(g)
    z = _tc_post(accp, g, dis, row(p["b3"]), p["Win0"], row(p["bin0"]),
                 p["Win1"], row(p["bin1"]), p["Wt1"], row(p["bt1"]),
                 p["Wt2"], row(p["bt2"])).reshape(C, 128)
    z7 = _tc_pooled(acnt.reshape(C, C), z,
                    p["W4"], row(p["b4"]),
                    p["W5"], row(p["b5"]), p["Wfc1"], row(p["bfc1"]),
                    p["Wbv0"], row(p["bbv0"]), p["Wbv1"], row(p["bbv1"]))
    k = z7.reshape(N, CN)
    return _tc_head(k, p["Wfc2"], row(p["bfc2"]), p["Wfc3"], row(p["bfc3"]),
                    p["Wav0"], row(p["bav0"]), p["Wav1"], row(p["bav1"]))


# speedup vs baseline: 13.0854x; 1.6236x over previous
import functools

import jax
import jax.numpy as jnp
from jax import lax
from jax.experimental import pallas as pl
from jax.experimental.pallas import tpu as pltpu
from jax.experimental.pallas import tpu_sc as plsc

N = 10000
E = 320000
D = 128
GNN = 1000
CN = 100
C = 1000
CC = C * C
EPS = 1e-5

NC = 2
NS = 16
NW = NC * NS
ERP = 2560
RPW = ERP // NW
RPC = ERP // NS
EPAD = ERP * 128 - E
NP = N + 16
RT = 624
DT = 624
CH = 40

_f32 = jnp.float32
_i32 = jnp.int32

_sc_mesh = plsc.VectorSubcoreMesh(
    core_axis_name="c", subcore_axis_name="s", num_cores=NC, num_subcores=NS)


def _elu(v):
    return jnp.where(v > 0, v, jnp.exp(v) - 1.0)


@functools.partial(
    pl.kernel,
    out_type=(jax.ShapeDtypeStruct((NC * N,), _f32),
              jax.ShapeDtypeStruct((NC * CC,), _f32)),
    mesh=_sc_mesh,
    scratch_types=[
        pltpu.VMEM_SHARED((NP,), _f32),
        pltpu.VMEM_SHARED((CC + 16,), _f32),
        pltpu.VMEM((RPW, 128), _i32),
        pltpu.VMEM((RPW, 128), _i32),
        pltpu.VMEM((RPW, 128), _i32),
        pltpu.VMEM((128,), _f32),
        pltpu.VMEM((2000,), _f32),
    ],
)
def _sc_counts(src2d, dst2d, zeros1d, deg_out, acnt_out,
               deg_sh, acnt_sh, sbuf2d, dbuf2d, pbuf2d, ones_b, zbuf):
    cid = lax.axis_index("c")
    sid = lax.axis_index("s")
    wid = sid * NC + cid

    pltpu.sync_copy(zeros1d.at[pl.ds(0, 2000)], zbuf)

    doff = pl.multiple_of(sid * DT, 8)
    pltpu.sync_copy(zbuf.at[pl.ds(0, DT)], deg_sh.at[pl.ds(doff, DT)])

    @pl.when(sid == 0)
    def _():
        pltpu.sync_copy(zbuf.at[pl.ds(0, N - NS * DT)],
                        deg_sh.at[pl.ds(NS * DT, N - NS * DT)])

    @pl.loop(0, 32)
    def _(t):
        ci = t * NS + sid

        @pl.when(ci < 500)
        def _():
            pltpu.sync_copy(zbuf, acnt_sh.at[pl.ds(ci * 2000, 2000)])

    for l in range(8):
        ones_b[pl.ds(l * 16, 16)] = jnp.ones((16,), _f32)

    base = pl.multiple_of(wid * RPW, 8)
    pltpu.sync_copy(src2d.at[pl.ds(base, RPW)], sbuf2d)
    pltpu.sync_copy(dst2d.at[pl.ds(base, RPW)], dbuf2d)

    @pl.loop(0, RPW)
    def _(j):
        for l in range(8):
            sv = sbuf2d[j, pl.ds(l * 16, 16)]
            dv = dbuf2d[j, pl.ds(l * 16, 16)]
            cs = lax.div(sv, GNN) * CN + lax.rem(sv, CN)
            cd = lax.div(dv, GNN) * CN + lax.rem(dv, CN)
            pbuf2d[j, pl.ds(l * 16, 16)] = cd * C + cs

    plsc.subcore_barrier()

    @pl.loop(0, RPW)
    def _(j):
        pltpu.sync_copy(ones_b, deg_sh.at[dbuf2d.at[j]], add=True)
        pltpu.sync_copy(ones_b, acnt_sh.at[pbuf2d.at[j]], add=True)

    plsc.subcore_barrier()

    cbase_n = cid * N
    pltpu.sync_copy(deg_sh.at[pl.ds(doff, DT)], zbuf.at[pl.ds(0, DT)])
    pltpu.sync_copy(zbuf.at[pl.ds(0, DT)],
                    deg_out.at[pl.ds(cbase_n + doff, DT)])

    @pl.when(sid == 0)
    def _():
        pltpu.sync_copy(deg_sh.at[pl.ds(NS * DT, N - NS * DT)],
                        zbuf.at[pl.ds(DT, N - NS * DT)])
        pltpu.sync_copy(zbuf.at[pl.ds(DT, N - NS * DT)],
                        deg_out.at[pl.ds(cbase_n + NS * DT, N - NS * DT)])

    cbase_a = cid * CC

    @pl.loop(0, 32)
    def _(t):
        ci = t * NS + sid

        @pl.when(ci < 500)
        def _():
            pltpu.sync_copy(acnt_sh.at[pl.ds(ci * 2000, 2000)], zbuf)
            pltpu.sync_copy(zbuf,
                            acnt_out.at[pl.ds(cbase_a + ci * 2000, 2000)])


@functools.partial(
    pl.kernel,
    out_type=jax.ShapeDtypeStruct((NC, N, D), _f32),
    mesh=_sc_mesh,
    scratch_types=[
        pltpu.VMEM_SHARED((NP, D), _f32),
        pltpu.VMEM((CH, 128), _i32),
        pltpu.VMEM((CH, 128), _i32),
        pltpu.VMEM((128, D), _f32),
        pltpu.VMEM((128, D), _f32),
        pltpu.SemaphoreType.DMA,
        pltpu.SemaphoreType.DMA,
    ],
)
def _sc_scatter(g_hbm, src2d, dst2d, zrows, acc_out,
                acc_sh, sbuf2d, dbuf2d, rows0, rows1, semg0, semg1):
    cid = lax.axis_index("c")
    sid = lax.axis_index("s")
    wid = sid * NC + cid

    base = pl.multiple_of(wid * RPW, 8)
    off = pl.multiple_of(sid * RT, 8)
    rows = (rows0, rows1)
    semg = (semg0, semg1)

    pltpu.sync_copy(zrows, rows0)

    @pl.loop(0, 4)
    def _(q):
        pltpu.sync_copy(rows0, acc_sh.at[pl.ds(off + q * 128, 128)])

    pltpu.sync_copy(rows0.at[pl.ds(0, RT - 512)],
                    acc_sh.at[pl.ds(off + 512, RT - 512)])

    @pl.when(sid == 0)
    def _():
        pltpu.sync_copy(rows0.at[pl.ds(0, N - NS * RT)],
                        acc_sh.at[pl.ds(NS * RT, N - NS * RT)])

    plsc.subcore_barrier()

    for c in range(RPW // CH):
        cb = pl.multiple_of(base + c * CH, 8)
        pltpu.sync_copy(src2d.at[pl.ds(cb, CH)], sbuf2d)
        pltpu.sync_copy(dst2d.at[pl.ds(cb, CH)], dbuf2d)

        pltpu.async_copy(g_hbm.at[sbuf2d.at[0]], rows0, semg0)

        @pl.loop(0, CH // 2)
        def _(t):
            for b in range(2):
                j = t * 2 + b
                pltpu.make_async_copy(g_hbm.at[sbuf2d.at[j]], rows[b],
                                      semg[b]).wait()

                @pl.when(j + 1 < CH)
                def _():
                    pltpu.async_copy(g_hbm.at[sbuf2d.at[j + 1]], rows[1 - b],
                                     semg[1 - b])

                pltpu.sync_copy(rows[b], acc_sh.at[dbuf2d.at[j]], add=True)

    plsc.subcore_barrier()

    pltpu.sync_copy(acc_sh.at[pl.ds(off, RT)],
                    acc_out.at[cid, pl.ds(off, RT)])

    @pl.when(sid == 0)
    def _():
        pltpu.sync_copy(acc_sh.at[pl.ds(NS * RT, N - NS * RT)],
                        acc_out.at[cid, pl.ds(NS * RT, N - NS * RT)])


_GB = 1000


def _acc_sum(ap_ref):
    return ap_ref[0] + ap_ref[1]


def _tc_prep_body(x_ref, w_ref, dis_ref, g_ref):
    g_ref[...] = dis_ref[...] * jnp.dot(x_ref[...], w_ref[...],
                                        preferred_element_type=_f32)


_tc_prep = pl.pallas_call(
    _tc_prep_body,
    grid=(N // _GB,),
    in_specs=[
        pl.BlockSpec((_GB, D), lambda i: (i, 0)),
        pl.BlockSpec((D, D), lambda i: (0, 0)),
        pl.BlockSpec((_GB, 1), lambda i: (i, 0)),
    ],
    out_specs=pl.BlockSpec((_GB, D), lambda i: (i, 0)),
    out_shape=jax.ShapeDtypeStruct((N, D), _f32),
)


def _tc_layer_body(ap_ref, g_ref, dis_ref, b_ref, w_ref, o_ref):
    dis = dis_ref[...]
    g = g_ref[...]
    h = dis * (_acc_sum(ap_ref) + g) + b_ref[...]
    h = _elu(h)
    o_ref[...] = dis * jnp.dot(h, w_ref[...], preferred_element_type=_f32)


_tc_layer = pl.pallas_call(
    _tc_layer_body,
    grid=(N // _GB,),
    in_specs=[
        pl.BlockSpec((NC, _GB, D), lambda i: (0, i, 0)),
        pl.BlockSpec((_GB, D), lambda i: (i, 0)),
        pl.BlockSpec((_GB, 1), lambda i: (i, 0)),
        pl.BlockSpec((1, D), lambda i: (0, 0)),
        pl.BlockSpec((D, D), lambda i: (0, 0)),
    ],
    out_specs=pl.BlockSpec((_GB, D), lambda i: (i, 0)),
    out_shape=jax.ShapeDtypeStruct((N, D), _f32),
)


def _tc_post_body(ap_ref, g_ref, dis_ref, b3_ref, win0_ref, bin0_ref,
                  win1_ref, bin1_ref, wt1_ref, bt1_ref, wt2_ref, bt2_ref,
                  z_ref):
    dis = dis_ref[...]
    h = dis * (_acc_sum(ap_ref) + g_ref[...]) + b3_ref[...]
    h = _elu(jnp.dot(h, win0_ref[...], preferred_element_type=_f32)
             + bin0_ref[...])
    h = _elu(jnp.dot(h, win1_ref[...], preferred_element_type=_f32)
             + bin1_ref[...])
    mean = jnp.mean(h, axis=0, keepdims=True)
    var = jnp.mean(h * h, axis=0, keepdims=True) - mean * mean
    y = (h - mean) * lax.rsqrt(var + EPS)
    xp = jnp.mean(y.reshape(GNN // CN, CN, D), axis=0)
    z = _elu(jnp.dot(xp, wt1_ref[...], preferred_element_type=_f32)
             + bt1_ref[...])
    z_ref[0] = _elu(jnp.dot(z, wt2_ref[...], preferred_element_type=_f32)
                    + bt2_ref[...])


_tc_post = pl.pallas_call(
    _tc_post_body,
    grid=(N // _GB,),
    in_specs=[
        pl.BlockSpec((NC, _GB, D), lambda i: (0, i, 0)),
        pl.BlockSpec((_GB, D), lambda i: (i, 0)),
        pl.BlockSpec((_GB, 1), lambda i: (i, 0)),
        pl.BlockSpec((1, D), lambda i: (0, 0)),
        pl.BlockSpec((D, D), lambda i: (0, 0)),
        pl.BlockSpec((1, D), lambda i: (0, 0)),
        pl.BlockSpec((D, D), lambda i: (0, 0)),
        pl.BlockSpec((1, D), lambda i: (0, 0)),
        pl.BlockSpec((D, 128), lambda i: (0, 0)),
        pl.BlockSpec((1, 128), lambda i: (0, 0)),
        pl.BlockSpec((128, 128), lambda i: (0, 0)),
        pl.BlockSpec((1, 128), lambda i: (0, 0)),
    ],
    out_specs=pl.BlockSpec((1, CN, 128), lambda i: (i, 0, 0)),
    out_shape=jax.ShapeDtypeStruct((N // _GB, CN, 128), _f32),
)


def _tc_pooled_body(at_ref, z_ref, w4_ref, b4_ref, w5_ref, b5_ref,
                    wfc1_ref, bfc1_ref, wbv0_ref, bbv0_ref, wbv1_ref,
                    bbv1_ref, o_ref):
    cnt = at_ref[0] + at_ref[1]
    r = lax.broadcasted_iota(_i32, (C, C), 0)
    cix = lax.broadcasted_iota(_i32, (C, C), 1)
    at = jnp.where((cnt > 0) & (r != cix), 1.0, 0.0).astype(_f32)
    deg2 = jnp.sum(at, axis=1, keepdims=True) + 1.0
    dis2 = lax.rsqrt(deg2)
    ateye = at + jnp.where(r == cix, 1.0, 0.0).astype(_f32)

    def pconv(zz, w, b):
        v = dis2 * jnp.dot(zz, w, preferred_element_type=_f32)
        return dis2 * jnp.dot(ateye, v, preferred_element_type=_f32) + b

    z = _elu(pconv(z_ref[...], w4_ref[...], b4_ref[...]))
    z = _elu(pconv(z, w5_ref[...], b5_ref[...]))
    z = _elu(jnp.dot(z, wfc1_ref[...], preferred_element_type=_f32)
             + bfc1_ref[...])
    z = _elu(jnp.dot(z, wbv0_ref[...], preferred_element_type=_f32)
             + bbv0_ref[...])
    o_ref[...] = _elu(jnp.dot(z, wbv1_ref[...], preferred_element_type=_f32)
                      + bbv1_ref[...])


_tc_pooled = pl.pallas_call(
    _tc_pooled_body,
    out_shape=jax.ShapeDtypeStruct((C, C), _f32),
)


def _tc_head_body(k_ref, wfc2_ref, bfc2_ref, wfc3_ref, bfc3_ref,
                  wav0_ref, bav0_ref, wav1_ref, bav1_ref, o_ref):
    k = _elu(jnp.dot(k_ref[...], wfc2_ref[...], preferred_element_type=_f32)
             + bfc2_ref[...])
    k = jnp.dot(k, wfc3_ref[...], preferred_element_type=_f32) + bfc3_ref[...]
    k = _elu(jnp.dot(k, wav0_ref[...], preferred_element_type=_f32)
             + bav0_ref[...])
    o_ref[...] = _elu(jnp.dot(k, wav1_ref[...], preferred_element_type=_f32)
                      + bav1_ref[...])


_tc_head = pl.pallas_call(
    _tc_head_body,
    grid=(N // _GB,),
    in_specs=[
        pl.BlockSpec((_GB, CN), lambda i: (i, 0)),
        pl.BlockSpec((CN, 256), lambda i: (0, 0)),
        pl.BlockSpec((1, 256), lambda i: (0, 0)),
        pl.BlockSpec((256, 128), lambda i: (0, 0)),
        pl.BlockSpec((1, 128), lambda i: (0, 0)),
        pl.BlockSpec((128, 128), lambda i: (0, 0)),
        pl.BlockSpec((1, 128), lambda i: (0, 0)),
        pl.BlockSpec((128, 128), lambda i: (0, 0)),
        pl.BlockSpec((1, 128), lambda i: (0, 0)),
    ],
    out_specs=pl.BlockSpec((_GB, 128), lambda i: (i, 0)),
    out_shape=jax.ShapeDtypeStruct((N, 128), _f32),
)


def kernel(x, adj, num_graphs, in_batch, cluster, params):
    p = params
    src2d = jnp.concatenate(
        [adj[0].astype(_i32), jnp.zeros((EPAD,), _i32)]).reshape(ERP, 128)
    dst2d = jnp.concatenate(
        [adj[1].astype(_i32), jnp.full((EPAD,), N, _i32)]).reshape(ERP, 128)
    zeros1d = jnp.zeros((8000,), _f32)
    zrows = jnp.zeros((128, D), _f32)

    degp, acnt = _sc_counts(src2d, dst2d, zeros1d)
    deg = degp.reshape(NC, N).sum(axis=0)
    dis = lax.rsqrt(deg + 1.0)[:, None]

    def row(b):
        return b.reshape(1, -1)

    def scatter(g):
        return _sc_scatter(g, src2d, dst2d, zrows)

    g = _tc_prep(x, p["W1"], dis)
    accp = scatter(g)
    g = _tc_layer(accp, g, dis, row(p["b1"]), p["W2"])
    accp = scatter(g)
    g = _tc_layer(accp, g, dis, row(p["b2"]), p["W3"])
    accp = scatter(g)
    z = _tc_post(accp, g, dis, row(p["b3"]), p["Win0"], row(p["bin0"]),
                 p["Win1"], row(p["bin1"]), p["Wt1"], row(p["bt1"]),
                 p["Wt2"], row(p["bt2"])).reshape(C, 128)
    z7 = _tc_pooled(acnt.reshape(NC, C, C), z,
                    p["W4"], row(p["b4"]),
                    p["W5"], row(p["b5"]), p["Wfc1"], row(p["bfc1"]),
                    p["Wbv0"], row(p["bbv0"]), p["Wbv1"], row(p["bbv1"]))
    k = z7.reshape(N, CN)
    return _tc_head(k, p["Wfc2"], row(p["bfc2"]), p["Wfc3"], row(p["bfc3"]),
                    p["Wav0"], row(p["bav0"]), p["Wav1"], row(p["bav1"]))


# speedup vs baseline: 13.1166x; 1.0024x over previous
import functools

import jax
import jax.numpy as jnp
from jax import lax
from jax.experimental import pallas as pl
from jax.experimental.pallas import tpu as pltpu
from jax.experimental.pallas import tpu_sc as plsc

N = 10000
E = 320000
D = 128
GNN = 1000
CN = 100
C = 1000
CC = C * C
EPS = 1e-5

NC = 2
NS = 16
NW = NC * NS
ERP = 2560
RPW = ERP // NW
RPC = ERP // NS
EPAD = ERP * 128 - E
NP = N + 16
RT = 624
DT = 624
CH = 40

_f32 = jnp.float32
_i32 = jnp.int32

_sc_mesh = plsc.VectorSubcoreMesh(
    core_axis_name="c", subcore_axis_name="s", num_cores=NC, num_subcores=NS)


def _elu(v):
    return jnp.where(v > 0, v, jnp.exp(v) - 1.0)


@functools.partial(
    pl.kernel,
    out_type=(jax.ShapeDtypeStruct((NC * N,), _f32),
              jax.ShapeDtypeStruct((NC * CC,), _f32)),
    mesh=_sc_mesh,
    scratch_types=[
        pltpu.VMEM_SHARED((NP,), _f32),
        pltpu.VMEM_SHARED((CC + 16,), _f32),
        pltpu.VMEM((RPW, 128), _i32),
        pltpu.VMEM((RPW, 128), _i32),
        pltpu.VMEM((RPW, 128), _i32),
        pltpu.VMEM((128,), _f32),
        pltpu.VMEM((8000,), _f32),
    ],
)
def _sc_counts(src2d, dst2d, zeros1d, deg_out, acnt_out,
               deg_sh, acnt_sh, sbuf2d, dbuf2d, pbuf2d, ones_b, zbuf):
    cid = lax.axis_index("c")
    sid = lax.axis_index("s")
    wid = sid * NC + cid

    pltpu.sync_copy(zeros1d.at[pl.ds(0, 8000)], zbuf)

    doff = pl.multiple_of(sid * DT, 8)
    pltpu.sync_copy(zbuf.at[pl.ds(0, DT)], deg_sh.at[pl.ds(doff, DT)])

    @pl.when(sid == 0)
    def _():
        pltpu.sync_copy(zbuf.at[pl.ds(0, N - NS * DT)],
                        deg_sh.at[pl.ds(NS * DT, N - NS * DT)])

    @pl.loop(0, 8)
    def _(t):
        ci = t * NS + sid

        @pl.when(ci < 125)
        def _():
            pltpu.sync_copy(zbuf, acnt_sh.at[pl.ds(ci * 8000, 8000)])

    for l in range(8):
        ones_b[pl.ds(l * 16, 16)] = jnp.ones((16,), _f32)

    base = pl.multiple_of(wid * RPW, 8)
    pltpu.sync_copy(src2d.at[pl.ds(base, RPW)], sbuf2d)
    pltpu.sync_copy(dst2d.at[pl.ds(base, RPW)], dbuf2d)

    @pl.loop(0, RPW)
    def _(j):
        for l in range(8):
            sv = sbuf2d[j, pl.ds(l * 16, 16)]
            dv = dbuf2d[j, pl.ds(l * 16, 16)]
            cs = lax.div(sv, GNN) * CN + lax.rem(sv, CN)
            cd = lax.div(dv, GNN) * CN + lax.rem(dv, CN)
            pbuf2d[j, pl.ds(l * 16, 16)] = cd * C + cs

    plsc.subcore_barrier()

    @pl.loop(0, RPW)
    def _(j):
        pltpu.sync_copy(ones_b, deg_sh.at[dbuf2d.at[j]], add=True)
        pltpu.sync_copy(ones_b, acnt_sh.at[pbuf2d.at[j]], add=True)

    plsc.subcore_barrier()

    cbase_n = cid * N
    pltpu.sync_copy(deg_sh.at[pl.ds(doff, DT)], zbuf.at[pl.ds(0, DT)])
    pltpu.sync_copy(zbuf.at[pl.ds(0, DT)],
                    deg_out.at[pl.ds(cbase_n + doff, DT)])

    @pl.when(sid == 0)
    def _():
        pltpu.sync_copy(deg_sh.at[pl.ds(NS * DT, N - NS * DT)],
                        zbuf.at[pl.ds(DT, N - NS * DT)])
        pltpu.sync_copy(zbuf.at[pl.ds(DT, N - NS * DT)],
                        deg_out.at[pl.ds(cbase_n + NS * DT, N - NS * DT)])

    cbase_a = cid * CC

    @pl.loop(0, 8)
    def _(t):
        ci = t * NS + sid

        @pl.when(ci < 125)
        def _():
            pltpu.sync_copy(acnt_sh.at[pl.ds(ci * 8000, 8000)], zbuf)
            pltpu.sync_copy(zbuf,
                            acnt_out.at[pl.ds(cbase_a + ci * 8000, 8000)])


@functools.partial(
    pl.kernel,
    out_type=jax.ShapeDtypeStruct((NC, N, D), _f32),
    mesh=_sc_mesh,
    scratch_types=[
        pltpu.VMEM_SHARED((NP, D), _f32),
        pltpu.VMEM((CH, 128), _i32),
        pltpu.VMEM((CH, 128), _i32),
        pltpu.VMEM((128, D), _f32),
        pltpu.VMEM((128, D), _f32),
        pltpu.SemaphoreType.DMA,
        pltpu.SemaphoreType.DMA,
    ],
)
def _sc_scatter(g_hbm, src2d, dst2d, zrows, acc_out,
                acc_sh, sbuf2d, dbuf2d, rows0, rows1, semg0, semg1):
    cid = lax.axis_index("c")
    sid = lax.axis_index("s")
    wid = sid * NC + cid

    base = pl.multiple_of(wid * RPW, 8)
    off = pl.multiple_of(sid * RT, 8)
    rows = (rows0, rows1)
    semg = (semg0, semg1)

    pltpu.sync_copy(zrows, rows0)

    @pl.loop(0, 4)
    def _(q):
        pltpu.sync_copy(rows0, acc_sh.at[pl.ds(off + q * 128, 128)])

    pltpu.sync_copy(rows0.at[pl.ds(0, RT - 512)],
                    acc_sh.at[pl.ds(off + 512, RT - 512)])

    @pl.when(sid == 0)
    def _():
        pltpu.sync_copy(rows0.at[pl.ds(0, N - NS * RT)],
                        acc_sh.at[pl.ds(NS * RT, N - NS * RT)])

    plsc.subcore_barrier()

    for c in range(RPW // CH):
        cb = pl.multiple_of(base + c * CH, 8)
        pltpu.sync_copy(src2d.at[pl.ds(cb, CH)], sbuf2d)
        pltpu.sync_copy(dst2d.at[pl.ds(cb, CH)], dbuf2d)

        pltpu.async_copy(g_hbm.at[sbuf2d.at[0]], rows0, semg0)

        @pl.loop(0, CH // 2)
        def _(t):
            for b in range(2):
                j = t * 2 + b
                pltpu.make_async_copy(g_hbm.at[sbuf2d.at[j]], rows[b],
                                      semg[b]).wait()

                @pl.when(j + 1 < CH)
                def _():
                    pltpu.async_copy(g_hbm.at[sbuf2d.at[j + 1]], rows[1 - b],
                                     semg[1 - b])

                pltpu.sync_copy(rows[b], acc_sh.at[dbuf2d.at[j]], add=True)

    plsc.subcore_barrier()

    pltpu.sync_copy(acc_sh.at[pl.ds(off, RT)],
                    acc_out.at[cid, pl.ds(off, RT)])

    @pl.when(sid == 0)
    def _():
        pltpu.sync_copy(acc_sh.at[pl.ds(NS * RT, N - NS * RT)],
                        acc_out.at[cid, pl.ds(NS * RT, N - NS * RT)])


_GB = 1000


def _acc_sum(ap_ref):
    return ap_ref[0] + ap_ref[1]


def _tc_prep_body(x_ref, w_ref, dis_ref, g_ref):
    g_ref[...] = dis_ref[...] * jnp.dot(x_ref[...], w_ref[...],
                                        preferred_element_type=_f32)


_tc_prep = pl.pallas_call(
    _tc_prep_body,
    grid=(N // _GB,),
    in_specs=[
        pl.BlockSpec((_GB, D), lambda i: (i, 0)),
        pl.BlockSpec((D, D), lambda i: (0, 0)),
        pl.BlockSpec((_GB, 1), lambda i: (i, 0)),
    ],
    out_specs=pl.BlockSpec((_GB, D), lambda i: (i, 0)),
    out_shape=jax.ShapeDtypeStruct((N, D), _f32),
)


def _tc_layer_body(ap_ref, g_ref, dis_ref, b_ref, w_ref, o_ref):
    dis = dis_ref[...]
    g = g_ref[...]
    h = dis * (_acc_sum(ap_ref) + g) + b_ref[...]
    h = _elu(h)
    o_ref[...] = dis * jnp.dot(h, w_ref[...], preferred_element_type=_f32)


_tc_layer = pl.pallas_call(
    _tc_layer_body,
    grid=(N // _GB,),
    in_specs=[
        pl.BlockSpec((NC, _GB, D), lambda i: (0, i, 0)),
        pl.BlockSpec((_GB, D), lambda i: (i, 0)),
        pl.BlockSpec((_GB, 1), lambda i: (i, 0)),
        pl.BlockSpec((1, D), lambda i: (0, 0)),
        pl.BlockSpec((D, D), lambda i: (0, 0)),
    ],
    out_specs=pl.BlockSpec((_GB, D), lambda i: (i, 0)),
    out_shape=jax.ShapeDtypeStruct((N, D), _f32),
)


def _tc_post_body(ap_ref, g_ref, dis_ref, b3_ref, win0_ref, bin0_ref,
                  win1_ref, bin1_ref, wt1_ref, bt1_ref, wt2_ref, bt2_ref,
                  z_ref):
    dis = dis_ref[...]
    h = dis * (_acc_sum(ap_ref) + g_ref[...]) + b3_ref[...]
    h = _elu(jnp.dot(h, win0_ref[...], preferred_element_type=_f32)
             + bin0_ref[...])
    h = _elu(jnp.dot(h, win1_ref[...], preferred_element_type=_f32)
             + bin1_ref[...])
    mean = jnp.mean(h, axis=0, keepdims=True)
    var = jnp.mean(h * h, axis=0, keepdims=True) - mean * mean
    y = (h - mean) * lax.rsqrt(var + EPS)
    xp = jnp.mean(y.reshape(GNN // CN, CN, D), axis=0)
    z = _elu(jnp.dot(xp, wt1_ref[...], preferred_element_type=_f32)
             + bt1_ref[...])
    z_ref[0] = _elu(jnp.dot(z, wt2_ref[...], preferred_element_type=_f32)
                    + bt2_ref[...])


_tc_post = pl.pallas_call(
    _tc_post_body,
    grid=(N // _GB,),
    in_specs=[
        pl.BlockSpec((NC, _GB, D), lambda i: (0, i, 0)),
        pl.BlockSpec((_GB, D), lambda i: (i, 0)),
        pl.BlockSpec((_GB, 1), lambda i: (i, 0)),
        pl.BlockSpec((1, D), lambda i: (0, 0)),
        pl.BlockSpec((D, D), lambda i: (0, 0)),
        pl.BlockSpec((1, D), lambda i: (0, 0)),
        pl.BlockSpec((D, D), lambda i: (0, 0)),
        pl.BlockSpec((1, D), lambda i: (0, 0)),
        pl.BlockSpec((D, 128), lambda i: (0, 0)),
        pl.BlockSpec((1, 128), lambda i: (0, 0)),
        pl.BlockSpec((128, 128), lambda i: (0, 0)),
        pl.BlockSpec((1, 128), lambda i: (0, 0)),
    ],
    out_specs=pl.BlockSpec((1, CN, 128), lambda i: (i, 0, 0)),
    out_shape=jax.ShapeDtypeStruct((N // _GB, CN, 128), _f32),
)


def _tc_pooled_body(at_ref, z_ref, w4_ref, b4_ref, w5_ref, b5_ref,
                    wfc1_ref, bfc1_ref, wbv0_ref, bbv0_ref, wbv1_ref,
                    bbv1_ref, o_ref):
    cnt = at_ref[0] + at_ref[1]
    r = lax.broadcasted_iota(_i32, (C, C), 0)
    cix = lax.broadcasted_iota(_i32, (C, C), 1)
    at = jnp.where((cnt > 0) & (r != cix), 1.0, 0.0).astype(_f32)
    deg2 = jnp.sum(at, axis=1, keepdims=True) + 1.0
    dis2 = lax.rsqrt(deg2)
    ateye = at + jnp.where(r == cix, 1.0, 0.0).astype(_f32)

    def pconv(zz, w, b):
        v = dis2 * jnp.dot(zz, w, preferred_element_type=_f32)
        return dis2 * jnp.dot(ateye, v, preferred_element_type=_f32) + b

    z = _elu(pconv(z_ref[...], w4_ref[...], b4_ref[...]))
    z = _elu(pconv(z, w5_ref[...], b5_ref[...]))
    z = _elu(jnp.dot(z, wfc1_ref[...], preferred_element_type=_f32)
             + bfc1_ref[...])
    z = _elu(jnp.dot(z, wbv0_ref[...], preferred_element_type=_f32)
             + bbv0_ref[...])
    o_ref[...] = _elu(jnp.dot(z, wbv1_ref[...], preferred_element_type=_f32)
                      + bbv1_ref[...])


_tc_pooled = pl.pallas_call(
    _tc_pooled_body,
    out_shape=jax.ShapeDtypeStruct((C, C), _f32),
)


def _tc_head_body(k_ref, wfc2_ref, bfc2_ref, wfc3_ref, bfc3_ref,
                  wav0_ref, bav0_ref, wav1_ref, bav1_ref, o_ref):
    k = _elu(jnp.dot(k_ref[...], wfc2_ref[...], preferred_element_type=_f32)
             + bfc2_ref[...])
    k = jnp.dot(k, wfc3_ref[...], preferred_element_type=_f32) + bfc3_ref[...]
    k = _elu(jnp.dot(k, wav0_ref[...], preferred_element_type=_f32)
             + bav0_ref[...])
    o_ref[...] = _elu(jnp.dot(k, wav1_ref[...], preferred_element_type=_f32)
                      + bav1_ref[...])


_tc_head = pl.pallas_call(
    _tc_head_body,
    grid=(N // _GB,),
    in_specs=[
        pl.BlockSpec((_GB, CN), lambda i: (i, 0)),
        pl.BlockSpec((CN, 256), lambda i: (0, 0)),
        pl.BlockSpec((1, 256), lambda i: (0, 0)),
        pl.BlockSpec((256, 128), lambda i: (0, 0)),
        pl.BlockSpec((1, 128), lambda i: (0, 0)),
        pl.BlockSpec((128, 128), lambda i: (0, 0)),
        pl.BlockSpec((1, 128), lambda i: (0, 0)),
        pl.BlockSpec((128, 128), lambda i: (0, 0)),
        pl.BlockSpec((1, 128), lambda i: (0, 0)),
    ],
    out_specs=pl.BlockSpec((_GB, 128), lambda i: (i, 0)),
    out_shape=jax.ShapeDtypeStruct((N, 128), _f32),
)


def kernel(x, adj, num_graphs, in_batch, cluster, params):
    p = params
    src2d = jnp.concatenate(
        [adj[0].astype(_i32), jnp.zeros((EPAD,), _i32)]).reshape(ERP, 128)
    dst2d = jnp.concatenate(
        [adj[1].astype(_i32), jnp.full((EPAD,), N, _i32)]).reshape(ERP, 128)
    zeros1d = jnp.zeros((8000,), _f32)
    zrows = jnp.zeros((128, D), _f32)

    degp, acnt = _sc_counts(src2d, dst2d, zeros1d)
    deg = degp.reshape(NC, N).sum(axis=0)
    dis = lax.rsqrt(deg + 1.0)[:, None]

    def row(b):
        return b.reshape(1, -1)

    def scatter(g):
        return _sc_scatter(g, src2d, dst2d, zrows)

    g = _tc_prep(x, p["W1"], dis)
    accp = scatter(g)
    g = _tc_layer(accp, g, dis, row(p["b1"]), p["W2"])
    accp = scatter(g)
    g = _tc_layer(accp, g, dis, row(p["b2"]), p["W3"])
    accp = scatter(g)
    z = _tc_post(accp, g, dis, row(p["b3"]), p["Win0"], row(p["bin0"]),
                 p["Win1"], row(p["bin1"]), p["Wt1"], row(p["bt1"]),
                 p["Wt2"], row(p["bt2"])).reshape(C, 128)
    z7 = _tc_pooled(acnt.reshape(NC, C, C), z,
                    p["W4"], row(p["b4"]),
                    p["W5"], row(p["b5"]), p["Wfc1"], row(p["bfc1"]),
                    p["Wbv0"], row(p["bbv0"]), p["Wbv1"], row(p["bbv1"]))
    k = z7.reshape(N, CN)
    return _tc_head(k, p["Wfc2"], row(p["bfc2"]), p["Wfc3"], row(p["bfc3"]),
                    p["Wav0"], row(p["bav0"]), p["Wav1"], row(p["bav1"]))


# speedup vs baseline: 14.2237x; 1.0844x over previous
import functools

import jax
import jax.numpy as jnp
from jax import lax
from jax.experimental import pallas as pl
from jax.experimental.pallas import tpu as pltpu
from jax.experimental.pallas import tpu_sc as plsc

N = 10000
E = 320000
D = 128
GNN = 1000
CN = 100
C = 1000
CC = C * C
EPS = 1e-5

NC = 2
NS = 16
NW = NC * NS
ERP = 2560
RPW = ERP // NW
RPC = ERP // NS
EPAD = ERP * 128 - E
NP = N + 16
RT = 624
DT = 624
CH = 40
FCH = 3
SCH = 1

_f32 = jnp.float32
_i32 = jnp.int32

_sc_mesh = plsc.VectorSubcoreMesh(
    core_axis_name="c", subcore_axis_name="s", num_cores=NC, num_subcores=NS)


def _elu(v):
    return jnp.where(v > 0, v, jnp.exp(v) - 1.0)


@functools.partial(
    pl.kernel,
    out_type=(jax.ShapeDtypeStruct((NC * N,), _f32),
              jax.ShapeDtypeStruct((NC * CC,), _f32)),
    mesh=_sc_mesh,
    scratch_types=[
        pltpu.VMEM_SHARED((NP,), _f32),
        pltpu.VMEM_SHARED((CC + 16,), _f32),
        pltpu.VMEM((RPW, 128), _i32),
        pltpu.VMEM((RPW, 128), _i32),
        pltpu.VMEM((RPW, 128), _i32),
        pltpu.VMEM((128,), _f32),
        pltpu.VMEM((8000,), _f32),
    ],
)
def _sc_counts(src2d, dst2d, zeros1d, deg_out, acnt_out,
               deg_sh, acnt_sh, sbuf2d, dbuf2d, pbuf2d, ones_b, zbuf):
    cid = lax.axis_index("c")
    sid = lax.axis_index("s")
    wid = sid * NC + cid

    pltpu.sync_copy(zeros1d.at[pl.ds(0, 8000)], zbuf)

    doff = pl.multiple_of(sid * DT, 8)
    pltpu.sync_copy(zbuf.at[pl.ds(0, DT)], deg_sh.at[pl.ds(doff, DT)])

    @pl.when(sid == 0)
    def _():
        pltpu.sync_copy(zbuf.at[pl.ds(0, N - NS * DT)],
                        deg_sh.at[pl.ds(NS * DT, N - NS * DT)])

    @pl.loop(0, 8)
    def _(t):
        ci = t * NS + sid

        @pl.when(ci < 125)
        def _():
            pltpu.sync_copy(zbuf, acnt_sh.at[pl.ds(ci * 8000, 8000)])

    for l in range(8):
        ones_b[pl.ds(l * 16, 16)] = jnp.ones((16,), _f32)

    base = pl.multiple_of(wid * RPW, 8)
    pltpu.sync_copy(src2d.at[pl.ds(base, RPW)], sbuf2d)
    pltpu.sync_copy(dst2d.at[pl.ds(base, RPW)], dbuf2d)

    @pl.loop(0, RPW)
    def _(j):
        for l in range(8):
            sv = sbuf2d[j, pl.ds(l * 16, 16)]
            dv = dbuf2d[j, pl.ds(l * 16, 16)]
            cs = lax.div(sv, GNN) * CN + lax.rem(sv, CN)
            cd = lax.div(dv, GNN) * CN + lax.rem(dv, CN)
            pbuf2d[j, pl.ds(l * 16, 16)] = cd * C + cs

    plsc.subcore_barrier()

    @pl.loop(0, RPW)
    def _(j):
        pltpu.sync_copy(ones_b, deg_sh.at[dbuf2d.at[j]], add=True)
        pltpu.sync_copy(ones_b, acnt_sh.at[pbuf2d.at[j]], add=True)

    plsc.subcore_barrier()

    cbase_n = cid * N
    pltpu.sync_copy(deg_sh.at[pl.ds(doff, DT)], zbuf.at[pl.ds(0, DT)])
    pltpu.sync_copy(zbuf.at[pl.ds(0, DT)],
                    deg_out.at[pl.ds(cbase_n + doff, DT)])

    @pl.when(sid == 0)
    def _():
        pltpu.sync_copy(deg_sh.at[pl.ds(NS * DT, N - NS * DT)],
                        zbuf.at[pl.ds(DT, N - NS * DT)])
        pltpu.sync_copy(zbuf.at[pl.ds(DT, N - NS * DT)],
                        deg_out.at[pl.ds(cbase_n + NS * DT, N - NS * DT)])

    cbase_a = cid * CC

    @pl.loop(0, 8)
    def _(t):
        ci = t * NS + sid

        @pl.when(ci < 125)
        def _():
            pltpu.sync_copy(acnt_sh.at[pl.ds(ci * 8000, 8000)], zbuf)
            pltpu.sync_copy(zbuf,
                            acnt_out.at[pl.ds(cbase_a + ci * 8000, 8000)])


@functools.partial(
    pl.kernel,
    out_type=jax.ShapeDtypeStruct((NC, N, D), _f32),
    mesh=_sc_mesh,
    scratch_types=[
        pltpu.VMEM_SHARED((NP, D), _f32),
        pltpu.VMEM((CH, 128), _i32),
        pltpu.VMEM((CH, 128), _i32),
        pltpu.VMEM((128, D), _f32),
        pltpu.VMEM((128, D), _f32),
        pltpu.SemaphoreType.DMA,
        pltpu.SemaphoreType.DMA,
    ],
)
def _sc_scatter(g_hbm, src2d, dst2d, zrows, acc_out,
                acc_sh, sbuf2d, dbuf2d, rows0, rows1, semg0, semg1):
    cid = lax.axis_index("c")
    sid = lax.axis_index("s")

    off = pl.multiple_of(sid * RT, 8)
    rows = (rows0, rows1)
    semg = (semg0, semg1)

    pltpu.sync_copy(zrows, rows0)

    @pl.loop(0, 4)
    def _(q):
        pltpu.sync_copy(rows0, acc_sh.at[pl.ds(off + q * 128, 128)])

    pltpu.sync_copy(rows0.at[pl.ds(0, RT - 512)],
                    acc_sh.at[pl.ds(off + 512, RT - 512)])

    @pl.when(sid == 0)
    def _():
        pltpu.sync_copy(rows0.at[pl.ds(0, N - NS * RT)],
                        acc_sh.at[pl.ds(NS * RT, N - NS * RT)])

    plsc.subcore_barrier()

    def _chunks(base0, nch):
        for c in range(nch):
            cb = pl.multiple_of(base0 + c * CH, 8)
            pltpu.sync_copy(src2d.at[pl.ds(cb, CH)], sbuf2d)
            pltpu.sync_copy(dst2d.at[pl.ds(cb, CH)], dbuf2d)

            pltpu.async_copy(g_hbm.at[sbuf2d.at[0]], rows0, semg0)

            @pl.loop(0, CH // 2)
            def _(t):
                for b in range(2):
                    j = t * 2 + b
                    pltpu.make_async_copy(g_hbm.at[sbuf2d.at[j]], rows[b],
                                          semg[b]).wait()

                    @pl.when(j + 1 < CH)
                    def _():
                        pltpu.async_copy(g_hbm.at[sbuf2d.at[j + 1]],
                                         rows[1 - b], semg[1 - b])

                    pltpu.sync_copy(rows[b], acc_sh.at[dbuf2d.at[j]],
                                    add=True)

    @pl.when(cid == 0)
    def _():
        _chunks(sid * (FCH * CH), FCH)

    @pl.when(cid == 1)
    def _():
        _chunks(NS * FCH * CH + sid * (SCH * CH), SCH)

    plsc.subcore_barrier()

    pltpu.sync_copy(acc_sh.at[pl.ds(off, RT)],
                    acc_out.at[cid, pl.ds(off, RT)])

    @pl.when(sid == 0)
    def _():
        pltpu.sync_copy(acc_sh.at[pl.ds(NS * RT, N - NS * RT)],
                        acc_out.at[cid, pl.ds(NS * RT, N - NS * RT)])


_GB = 1000


def _acc_sum(ap_ref):
    return ap_ref[0] + ap_ref[1]


def _tc_prep_body(x_ref, w_ref, dis_ref, g_ref):
    g_ref[...] = dis_ref[...] * jnp.dot(x_ref[...], w_ref[...],
                                        preferred_element_type=_f32)


_tc_prep = pl.pallas_call(
    _tc_prep_body,
    grid=(N // _GB,),
    in_specs=[
        pl.BlockSpec((_GB, D), lambda i: (i, 0)),
        pl.BlockSpec((D, D), lambda i: (0, 0)),
        pl.BlockSpec((_GB, 1), lambda i: (i, 0)),
    ],
    out_specs=pl.BlockSpec((_GB, D), lambda i: (i, 0)),
    out_shape=jax.ShapeDtypeStruct((N, D), _f32),
)


def _tc_layer_body(ap_ref, g_ref, dis_ref, b_ref, w_ref, o_ref):
    dis = dis_ref[...]
    g = g_ref[...]
    h = dis * (_acc_sum(ap_ref) + g) + b_ref[...]
    h = _elu(h)
    o_ref[...] = dis * jnp.dot(h, w_ref[...], preferred_element_type=_f32)


_tc_layer = pl.pallas_call(
    _tc_layer_body,
    grid=(N // _GB,),
    in_specs=[
        pl.BlockSpec((NC, _GB, D), lambda i: (0, i, 0)),
        pl.BlockSpec((_GB, D), lambda i: (i, 0)),
        pl.BlockSpec((_GB, 1), lambda i: (i, 0)),
        pl.BlockSpec((1, D), lambda i: (0, 0)),
        pl.BlockSpec((D, D), lambda i: (0, 0)),
    ],
    out_specs=pl.BlockSpec((_GB, D), lambda i: (i, 0)),
    out_shape=jax.ShapeDtypeStruct((N, D), _f32),
)


def _tc_post_body(ap_ref, g_ref, dis_ref, b3_ref, win0_ref, bin0_ref,
                  win1_ref, bin1_ref, wt1_ref, bt1_ref, wt2_ref, bt2_ref,
                  z_ref):
    dis = dis_ref[...]
    h = dis * (_acc_sum(ap_ref) + g_ref[...]) + b3_ref[...]
    h = _elu(jnp.dot(h, win0_ref[...], preferred_element_type=_f32)
             + bin0_ref[...])
    h = _elu(jnp.dot(h, win1_ref[...], preferred_element_type=_f32)
             + bin1_ref[...])
    mean = jnp.mean(h, axis=0, keepdims=True)
    var = jnp.mean(h * h, axis=0, keepdims=True) - mean * mean
    y = (h - mean) * lax.rsqrt(var + EPS)
    xp = jnp.mean(y.reshape(GNN // CN, CN, D), axis=0)
    z = _elu(jnp.dot(xp, wt1_ref[...], preferred_element_type=_f32)
             + bt1_ref[...])
    z_ref[0] = _elu(jnp.dot(z, wt2_ref[...], preferred_element_type=_f32)
                    + bt2_ref[...])


_tc_post = pl.pallas_call(
    _tc_post_body,
    grid=(N // _GB,),
    in_specs=[
        pl.BlockSpec((NC, _GB, D), lambda i: (0, i, 0)),
        pl.BlockSpec((_GB, D), lambda i: (i, 0)),
        pl.BlockSpec((_GB, 1), lambda i: (i, 0)),
        pl.BlockSpec((1, D), lambda i: (0, 0)),
        pl.BlockSpec((D, D), lambda i: (0, 0)),
        pl.BlockSpec((1, D), lambda i: (0, 0)),
        pl.BlockSpec((D, D), lambda i: (0, 0)),
        pl.BlockSpec((1, D), lambda i: (0, 0)),
        pl.BlockSpec((D, 128), lambda i: (0, 0)),
        pl.BlockSpec((1, 128), lambda i: (0, 0)),
        pl.BlockSpec((128, 128), lambda i: (0, 0)),
        pl.BlockSpec((1, 128), lambda i: (0, 0)),
    ],
    out_specs=pl.BlockSpec((1, CN, 128), lambda i: (i, 0, 0)),
    out_shape=jax.ShapeDtypeStruct((N // _GB, CN, 128), _f32),
)


def _tc_pooled_body(at_ref, z_ref, w4_ref, b4_ref, w5_ref, b5_ref,
                    wfc1_ref, bfc1_ref, wbv0_ref, bbv0_ref, wbv1_ref,
                    bbv1_ref, o_ref):
    cnt = at_ref[0] + at_ref[1]
    r = lax.broadcasted_iota(_i32, (C, C), 0)
    cix = lax.broadcasted_iota(_i32, (C, C), 1)
    at = jnp.where((cnt > 0) & (r != cix), 1.0, 0.0).astype(_f32)
    deg2 = jnp.sum(at, axis=1, keepdims=True) + 1.0
    dis2 = lax.rsqrt(deg2)
    ateye = at + jnp.where(r == cix, 1.0, 0.0).astype(_f32)

    def pconv(zz, w, b):
        v = dis2 * jnp.dot(zz, w, preferred_element_type=_f32)
        return dis2 * jnp.dot(ateye, v, preferred_element_type=_f32) + b

    z = _elu(pconv(z_ref[...], w4_ref[...], b4_ref[...]))
    z = _elu(pconv(z, w5_ref[...], b5_ref[...]))
    z = _elu(jnp.dot(z, wfc1_ref[...], preferred_element_type=_f32)
             + bfc1_ref[...])
    z = _elu(jnp.dot(z, wbv0_ref[...], preferred_element_type=_f32)
             + bbv0_ref[...])
    o_ref[...] = _elu(jnp.dot(z, wbv1_ref[...], preferred_element_type=_f32)
                      + bbv1_ref[...])


_tc_pooled = pl.pallas_call(
    _tc_pooled_body,
    out_shape=jax.ShapeDtypeStruct((C, C), _f32),
)


def _tc_head_body(k_ref, wfc2_ref, bfc2_ref, wfc3_ref, bfc3_ref,
                  wav0_ref, bav0_ref, wav1_ref, bav1_ref, o_ref):
    k = _elu(jnp.dot(k_ref[...], wfc2_ref[...], preferred_element_type=_f32)
             + bfc2_ref[...])
    k = jnp.dot(k, wfc3_ref[...], preferred_element_type=_f32) + bfc3_ref[...]
    k = _elu(jnp.dot(k, wav0_ref[...], preferred_element_type=_f32)
             + bav0_ref[...])
    o_ref[...] = _elu(jnp.dot(k, wav1_ref[...], preferred_element_type=_f32)
                      + bav1_ref[...])


_tc_head = pl.pallas_call(
    _tc_head_body,
    grid=(N // _GB,),
    in_specs=[
        pl.BlockSpec((_GB, CN), lambda i: (i, 0)),
        pl.BlockSpec((CN, 256), lambda i: (0, 0)),
        pl.BlockSpec((1, 256), lambda i: (0, 0)),
        pl.BlockSpec((256, 128), lambda i: (0, 0)),
        pl.BlockSpec((1, 128), lambda i: (0, 0)),
        pl.BlockSpec((128, 128), lambda i: (0, 0)),
        pl.BlockSpec((1, 128), lambda i: (0, 0)),
        pl.BlockSpec((128, 128), lambda i: (0, 0)),
        pl.BlockSpec((1, 128), lambda i: (0, 0)),
    ],
    out_specs=pl.BlockSpec((_GB, 128), lambda i: (i, 0)),
    out_shape=jax.ShapeDtypeStruct((N, 128), _f32),
)


def kernel(x, adj, num_graphs, in_batch, cluster, params):
    p = params
    src2d = jnp.concatenate(
        [adj[0].astype(_i32), jnp.zeros((EPAD,), _i32)]).reshape(ERP, 128)
    dst2d = jnp.concatenate(
        [adj[1].astype(_i32), jnp.full((EPAD,), N, _i32)]).reshape(ERP, 128)
    zeros1d = jnp.zeros((8000,), _f32)
    zrows = jnp.zeros((128, D), _f32)

    degp, acnt = _sc_counts(src2d, dst2d, zeros1d)
    deg = degp.reshape(NC, N).sum(axis=0)
    dis = lax.rsqrt(deg + 1.0)[:, None]

    def row(b):
        return b.reshape(1, -1)

    def scatter(g):
        return _sc_scatter(g, src2d, dst2d, zrows)

    g = _tc_prep(x, p["W1"], dis)
    accp = scatter(g)
    g = _tc_layer(accp, g, dis, row(p["b1"]), p["W2"])
    accp = scatter(g)
    g = _tc_layer(accp, g, dis, row(p["b2"]), p["W3"])
    accp = scatter(g)
    z = _tc_post(accp, g, dis, row(p["b3"]), p["Win0"], row(p["bin0"]),
                 p["Win1"], row(p["bin1"]), p["Wt1"], row(p["bt1"]),
                 p["Wt2"], row(p["bt2"])).reshape(C, 128)
    z7 = _tc_pooled(acnt.reshape(NC, C, C), z,
                    p["W4"], row(p["b4"]),
                    p["W5"], row(p["b5"]), p["Wfc1"], row(p["bfc1"]),
                    p["Wbv0"], row(p["bbv0"]), p["Wbv1"], row(p["bbv1"]))
    k = z7.reshape(N, CN)
    return _tc_head(k, p["Wfc2"], row(p["bfc2"]), p["Wfc3"], row(p["bfc3"]),
                    p["Wav0"], row(p["bav0"]), p["Wav1"], row(p["bav1"]))
